# baseline (speedup 1.0000x reference)
import functools

import jax
import jax.numpy as jnp
from jax import lax
from jax.experimental import pallas as pl
from jax.experimental.pallas import tpu as pltpu
from jax.experimental.pallas import tpu_sc as plsc

_B, _C, _H, _W = 8, 128, 224, 224
_HW = _H * _W
_V = 1000
_VP = 1024
_NC, _NS = 2, 16
_NW = _NC * _NS
_CBLK = 16
_NCB = _C // _CBLK
_BPW = _B * _NCB // _NW
_CH = 1792
_NCHUNK = _HW // _CH
_T = _BPW * _NCHUNK


def _transpose_table(tpad):
    def body(t_ref, o_ref):
        o_ref[...] = t_ref[...].T

    return pl.pallas_call(
        body, out_shape=jax.ShapeDtypeStruct((_C, _VP), jnp.float32)
    )(tpad)


def _sc_gather(table_t, idx):
    mesh = plsc.VectorSubcoreMesh(
        core_axis_name="c", subcore_axis_name="s",
        num_cores=_NC, num_subcores=_NS)

    @functools.partial(
        pl.kernel,
        out_type=jax.ShapeDtypeStruct((_B * _C * _HW,), jnp.float32),
        mesh=mesh,
        compiler_params=pltpu.CompilerParams(needs_layout_passes=False),
        scratch_types=[
            pltpu.VMEM((_CBLK * _VP,), jnp.float32),
            pltpu.VMEM((_CH,), jnp.int32),
            pltpu.VMEM((_CH,), jnp.int32),
            pltpu.VMEM((_CBLK * _CH,), jnp.float32),
            pltpu.VMEM((_CBLK * _CH,), jnp.float32),
            pltpu.SemaphoreType.DMA,
            pltpu.SemaphoreType.DMA,
            pltpu.SemaphoreType.DMA,
            pltpu.SemaphoreType.DMA,
            pltpu.SemaphoreType.DMA,
        ],
    )
    def k(tt_hbm, idx_hbm, out_hbm, lut_v, idx0, idx1, st0, st1,
          sem_lut, sem_i0, sem_i1, sem_o0, sem_o1):
        wid = lax.axis_index("s") * _NC + lax.axis_index("c")
        cblk = wid // (_NW // _NCB)
        bpair = wid % (_NW // _NCB)
        idx_bufs = (idx0, idx1)
        stages = (st0, st1)
        sem_is = (sem_i0, sem_i1)
        sem_os = (sem_o0, sem_o1)

        def idx_off(t):
            b = bpair * _BPW + t // _NCHUNK
            return b * _HW + (t % _NCHUNK) * _CH

        def out_off(t):
            b = bpair * _BPW + t // _NCHUNK
            return b * (_C * _HW) + (cblk * _CBLK) * _HW + (t % _NCHUNK) * _CH

        lut_copy = pltpu.async_copy(
            tt_hbm.at[pl.ds(cblk * (_CBLK * _VP), _CBLK * _VP)], lut_v,
            sem_lut)
        pltpu.async_copy(idx_hbm.at[pl.ds(idx_off(0), _CH)], idx0, sem_i0)
        pltpu.async_copy(idx_hbm.at[pl.ds(idx_off(1), _CH)], idx1, sem_i1)
        lut_copy.wait()

        def chunk_body(t, _):
            for s in (0, 1):
                te = t + s
                idx_v, stage_v = idx_bufs[s], stages[s]
                pltpu.make_async_copy(
                    idx_hbm.at[pl.ds(0, _CH)], idx_v, sem_is[s]).wait()
                @pl.when(te >= 2)
                def _drain_out():
                    pltpu.make_async_copy(
                        stage_v, out_hbm.at[pl.ds(0, _CBLK * _CH)],
                        sem_os[s]).wait()

                def pos_body(i, _):
                    iv = idx_v[pl.ds(i * 16, 16)]
                    for c in range(_CBLK):
                        stage_v[pl.ds(c * _CH + i * 16, 16)] = (
                            plsc.load_gather(lut_v, [iv + c * _VP]))
                    return 0

                lax.fori_loop(0, _CH // 16, pos_body, 0)
                obase = out_off(te)
                for c in range(_CBLK):
                    pltpu.async_copy(
                        stage_v.at[pl.ds(c * _CH, _CH)],
                        out_hbm.at[pl.ds(obase + c * _HW, _CH)], sem_os[s])
                @pl.when(te + 2 < _T)
                def _prefetch():
                    pltpu.async_copy(
                        idx_hbm.at[pl.ds(idx_off(te + 2), _CH)], idx_v,
                        sem_is[s])
            return 0

        lax.fori_loop(0, _T // 2, lambda u, c: chunk_body(u * 2, c), 0)
        for s in (0, 1):
            pltpu.make_async_copy(
                stages[s], out_hbm.at[pl.ds(0, _CBLK * _CH)], sem_os[s]).wait()

    return k(table_t, idx)


def kernel(x, table):
    idx = x.reshape(_B * _HW)
    tpad = jnp.zeros((_VP, _C), jnp.float32).at[:_V].set(table)
    table_t = _transpose_table(tpad).reshape(_C * _VP)
    out = _sc_gather(table_t, idx)
    return out.reshape(_B, _C, _H, _W)


# speedup vs baseline: 1.6410x; 1.6410x over previous
import functools

import jax
import jax.numpy as jnp
from jax import lax
from jax.experimental import pallas as pl
from jax.experimental.pallas import tpu as pltpu
from jax.experimental.pallas import tpu_sc as plsc

_B, _C, _H, _W = 8, 128, 224, 224
_HW = _H * _W
_V = 1000
_VP = 1024
_NC, _NS = 2, 16
_NW = _NC * _NS
_CBLK = 16
_NCB = _C // _CBLK
_BPW = _B * _NCB // _NW
_CH = 1792
_NCHUNK = _HW // _CH
_T = _BPW * _NCHUNK


def _transpose_table(tpad):
    def body(t_ref, o_ref):
        o_ref[...] = t_ref[...].T

    return pl.pallas_call(
        body, out_shape=jax.ShapeDtypeStruct((_C, _VP), jnp.float32)
    )(tpad)


def _sc_gather(table_t, idx):
    mesh = plsc.VectorSubcoreMesh(
        core_axis_name="c", subcore_axis_name="s",
        num_cores=_NC, num_subcores=_NS)

    @functools.partial(
        pl.kernel,
        out_type=jax.ShapeDtypeStruct((_B * _C * _HW,), jnp.float32),
        mesh=mesh,
        compiler_params=pltpu.CompilerParams(needs_layout_passes=False),
        scratch_types=[
            pltpu.VMEM((_CBLK * _VP,), jnp.float32),
            pltpu.VMEM((_CH,), jnp.int32),
            pltpu.VMEM((_CH,), jnp.int32),
            pltpu.VMEM((_CBLK * _CH,), jnp.float32),
            pltpu.VMEM((_CBLK * _CH,), jnp.float32),
            pltpu.SemaphoreType.DMA,
            pltpu.SemaphoreType.DMA,
            pltpu.SemaphoreType.DMA,
            pltpu.SemaphoreType.DMA,
            pltpu.SemaphoreType.DMA,
        ],
    )
    def k(tt_hbm, idx_hbm, out_hbm, lut_v, idx0, idx1, st0, st1,
          sem_lut, sem_i0, sem_i1, sem_o0, sem_o1):
        wid = lax.axis_index("s") * _NC + lax.axis_index("c")
        cblk = wid // (_NW // _NCB)
        bpair = wid % (_NW // _NCB)
        idx_bufs = (idx0, idx1)
        stages = (st0, st1)
        sem_is = (sem_i0, sem_i1)
        sem_os = (sem_o0, sem_o1)

        def idx_off(t):
            b = bpair * _BPW + t // _NCHUNK
            return b * _HW + (t % _NCHUNK) * _CH

        def out_off(t):
            b = bpair * _BPW + t // _NCHUNK
            return b * (_C * _HW) + (cblk * _CBLK) * _HW + (t % _NCHUNK) * _CH

        lut_copy = pltpu.async_copy(
            tt_hbm.at[pl.ds(cblk * (_CBLK * _VP), _CBLK * _VP)], lut_v,
            sem_lut)
        pltpu.async_copy(idx_hbm.at[pl.ds(idx_off(0), _CH)], idx0, sem_i0)
        pltpu.async_copy(idx_hbm.at[pl.ds(idx_off(1), _CH)], idx1, sem_i1)
        lut_copy.wait()

        def chunk_body(t, _):
            for s in (0, 1):
                te = t + s
                idx_v, stage_v = idx_bufs[s], stages[s]
                pltpu.make_async_copy(
                    idx_hbm.at[pl.ds(0, _CH)], idx_v, sem_is[s]).wait()
                @pl.when(te >= 2)
                def _drain_out():
                    pltpu.make_async_copy(
                        stage_v, out_hbm.at[pl.ds(0, _CBLK * _CH)],
                        sem_os[s]).wait()

                @plsc.parallel_loop(0, _CH // 16, unroll=4)
                def _pos_body(i):
                    iv = idx_v[pl.ds(i * 16, 16)]
                    for c in range(_CBLK):
                        stage_v[pl.ds(c * _CH + i * 16, 16)] = (
                            plsc.load_gather(lut_v, [iv + c * _VP]))
                obase = out_off(te)
                for c in range(_CBLK):
                    pltpu.async_copy(
                        stage_v.at[pl.ds(c * _CH, _CH)],
                        out_hbm.at[pl.ds(obase + c * _HW, _CH)], sem_os[s])
                @pl.when(te + 2 < _T)
                def _prefetch():
                    pltpu.async_copy(
                        idx_hbm.at[pl.ds(idx_off(te + 2), _CH)], idx_v,
                        sem_is[s])
            return 0

        lax.fori_loop(0, _T // 2, lambda u, c: chunk_body(u * 2, c), 0)
        for s in (0, 1):
            pltpu.make_async_copy(
                stages[s], out_hbm.at[pl.ds(0, _CBLK * _CH)], sem_os[s]).wait()

    return k(table_t, idx)


def kernel(x, table):
    idx = x.reshape(_B * _HW)
    tpad = jnp.zeros((_VP, _C), jnp.float32).at[:_V].set(table)
    table_t = _transpose_table(tpad).reshape(_C * _VP)
    out = _sc_gather(table_t, idx)
    return out.reshape(_B, _C, _H, _W)
